# baseline (speedup 1.0000x reference)
import functools

import jax
import jax.numpy as jnp
from jax.experimental import pallas as pl

N_QUERIES = 4096
N_KEYS = 100000
DIM = 128
TOPK = 100

BM = 1024
BN = 1024
KP = 100352


def _normalize_body(x_ref, o_ref):
    x = x_ref[...]
    n = jnp.sqrt(jnp.sum(x * x, axis=1, keepdims=True))
    o_ref[...] = x / jnp.maximum(n, 1e-8)


def _normalize(x, block_rows):
    rows = x.shape[0]
    return pl.pallas_call(
        _normalize_body,
        grid=(rows // block_rows,),
        in_specs=[pl.BlockSpec((block_rows, DIM), lambda i: (i, 0))],
        out_specs=pl.BlockSpec((block_rows, DIM), lambda i: (i, 0)),
        out_shape=jax.ShapeDtypeStruct((rows, DIM), jnp.float32),
    )(x)


def _matmul_body(q_ref, k_ref, o_ref):
    j = pl.program_id(1)
    s = jax.lax.dot_general(
        q_ref[...], k_ref[...],
        dimension_numbers=(((1,), (1,)), ((), ())),
        preferred_element_type=jnp.float32,
    )
    o_ref[...] = s

    @pl.when(j == (KP // BN) - 1)
    def _mask_tail():
        col = j * BN + jax.lax.broadcasted_iota(jnp.int32, (BM, BN), 1)
        o_ref[...] = jnp.where(col >= N_KEYS, jnp.float32(-2.0), s)


def _sim_matrix(qn, kn):
    return pl.pallas_call(
        _matmul_body,
        grid=(N_QUERIES // BM, KP // BN),
        in_specs=[
            pl.BlockSpec((BM, DIM), lambda i, j: (i, 0)),
            pl.BlockSpec((BN, DIM), lambda i, j: (j, 0)),
        ],
        out_specs=pl.BlockSpec((BM, BN), lambda i, j: (i, j)),
        out_shape=jax.ShapeDtypeStruct((N_QUERIES, KP), jnp.float32),
    )(qn, kn)


def kernel(queries, keys, k):
    q_n = jnp.linalg.norm(queries, axis=1, keepdims=True)
    k_n = jnp.linalg.norm(keys, axis=1, keepdims=True)
    qn = queries / jnp.maximum(q_n, 1e-8)
    kn = keys / jnp.maximum(k_n, 1e-8)
    kn_p = jnp.pad(kn, ((0, KP - N_KEYS), (0, 0)))
    sim = _sim_matrix(qn, kn_p)
    vals, idx = jax.lax.top_k(sim, TOPK)
    return vals, idx.astype(jnp.int32)


# speedup vs baseline: 4.6915x; 4.6915x over previous
import functools

import jax
import jax.numpy as jnp
from jax import lax
from jax.experimental import pallas as pl
from jax.experimental.pallas import tpu as pltpu
from jax.experimental.pallas import tpu_sc as plsc

N_QUERIES = 4096
N_KEYS = 100000
DIM = 128
TOPK = 100

BM = 1024
BN = 1024
KP = 100352

NW = 32
ROWS_PER_W = N_QUERIES // NW
NVEC = N_KEYS // 16
WVEC = 256
WARM = WVEC * 16
CAP = 10240
OUT_C = 512
PAD_VAL = -3.0
BS_ITERS = 17


def _matmul_body(q_ref, k_ref, o_ref):
    o_ref[...] = jax.lax.dot_general(
        q_ref[...], k_ref[...],
        dimension_numbers=(((1,), (1,)), ((), ())),
        preferred_element_type=jnp.float32,
    )


def _sim_matrix(qn, kn):
    return pl.pallas_call(
        _matmul_body,
        grid=(N_QUERIES // BM, KP // BN),
        in_specs=[
            pl.BlockSpec((BM, DIM), lambda i, j: (i, 0)),
            pl.BlockSpec((BN, DIM), lambda i, j: (j, 0)),
        ],
        out_specs=pl.BlockSpec((BM, BN), lambda i, j: (i, j)),
        out_shape=jax.ShapeDtypeStruct((N_QUERIES, KP), jnp.float32),
    )(qn, kn)


def _splat_f32(x):
    return jnp.full((16,), x, jnp.float32)


def _splat_total(x):
    s = plsc.cumsum(x)
    return plsc.cummax(lax.rev(s, (0,)))


def _splat_last(incl):
    return plsc.cummax(lax.rev(incl, (0,)))


def _thresh_search(read_vec, nvec, lo0, hi0):
    def bs_body(_, lohi):
        lo, hi = lohi
        mid = 0.5 * (lo + hi)
        def cnt_body(j, acc):
            m = read_vec(j) >= mid
            return acc + m.astype(jnp.int32)
        acc = lax.fori_loop(0, nvec, cnt_body, jnp.zeros((16,), jnp.int32))
        ok = _splat_total(acc) >= TOPK
        return jnp.where(ok, mid, lo), jnp.where(ok, hi, mid)
    lo, hi = lax.fori_loop(0, BS_ITERS, bs_body, (lo0, hi0))
    return lo


def _sc_select_body(sim_ref, vals_ref, idx_ref, row_v, cv, ci, ov, oi):
    cid = lax.axis_index("c")
    sid = lax.axis_index("s")
    wid = sid * 2 + cid
    iota = lax.iota(jnp.int32, 16)

    def do_row(r, _):
        row = wid * ROWS_PER_W + r
        pltpu.sync_copy(sim_ref.at[pl.ds(row * KP, N_KEYS)], row_v)

        t = _thresh_search(lambda j: row_v[pl.ds(j * 16, 16)], WVEC,
                           _splat_f32(-1.5), _splat_f32(1.5))

        def seed_body(j, _):
            ci[pl.ds(j * 16, 16)] = iota + j * 16
            return 0
        lax.fori_loop(0, WVEC, seed_body, 0)

        def filt_body(i, curv):
            v = row_v[pl.ds(i * 16, 16)]
            m = v >= t
            mi = m.astype(jnp.int32)
            incl = plsc.cumsum(mi)
            pos = curv + (incl - mi)
            pos = jnp.minimum(pos, CAP + 15)
            plsc.store_scatter(ci, [pos], iota + i * 16, mask=m)
            return curv + _splat_last(incl)
        curv = lax.fori_loop(WVEC, NVEC, filt_body,
                             jnp.full((16,), WARM, jnp.int32))

        def gath_body(j, _):
            ix = ci[pl.ds(j * 16, 16)]
            ixc = jnp.clip(ix, 0, N_KEYS - 1)
            v = plsc.load_gather(row_v, [ixc])
            valid = (iota + j * 16) < curv
            cv[pl.ds(j * 16, 16)] = jnp.where(valid, v, _splat_f32(PAD_VAL))
            return 0
        lax.fori_loop(0, CAP // 16, gath_body, 0)

        t2 = _thresh_search(lambda j: cv[pl.ds(j * 16, 16)], CAP // 16,
                            t, _splat_f32(1.5))

        def oclr_body(j, _):
            ov[pl.ds(j * 16, 16)] = _splat_f32(PAD_VAL)
            return 0
        lax.fori_loop(0, (OUT_C + 16) // 16, oclr_body, 0)

        def comp_body(j, ocur):
            v = cv[pl.ds(j * 16, 16)]
            ix = ci[pl.ds(j * 16, 16)]
            m = v >= t2
            mi = m.astype(jnp.int32)
            incl = plsc.cumsum(mi)
            pos = ocur + (incl - mi)
            pos = jnp.minimum(pos, OUT_C + 15)
            plsc.store_scatter(ov, [pos], v, mask=m)
            plsc.store_scatter(oi, [pos], ix, mask=m)
            return ocur + _splat_last(incl)
        lax.fori_loop(0, CAP // 16, comp_body, jnp.zeros((16,), jnp.int32))

        pltpu.sync_copy(ov.at[pl.ds(0, OUT_C)],
                        vals_ref.at[pl.ds(row * OUT_C, OUT_C)])
        pltpu.sync_copy(oi.at[pl.ds(0, OUT_C)],
                        idx_ref.at[pl.ds(row * OUT_C, OUT_C)])
        return 0

    lax.fori_loop(0, ROWS_PER_W, do_row, 0)


@functools.partial(
    pl.kernel,
    out_type=[
        jax.ShapeDtypeStruct((N_QUERIES * OUT_C,), jnp.float32),
        jax.ShapeDtypeStruct((N_QUERIES * OUT_C,), jnp.int32),
    ],
    mesh=plsc.VectorSubcoreMesh(
        core_axis_name="c", subcore_axis_name="s",
        num_cores=2, num_subcores=16),
    compiler_params=pltpu.CompilerParams(needs_layout_passes=False),
    scratch_types=[
        pltpu.VMEM((N_KEYS,), jnp.float32),
        pltpu.VMEM((CAP + 16,), jnp.float32),
        pltpu.VMEM((CAP + 16,), jnp.int32),
        pltpu.VMEM((OUT_C + 16,), jnp.float32),
        pltpu.VMEM((OUT_C + 16,), jnp.int32),
    ],
)
def _sc_select(sim_ref, vals_ref, idx_ref, row_v, cv, ci, ov, oi):
    _sc_select_body(sim_ref, vals_ref, idx_ref, row_v, cv, ci, ov, oi)


def kernel(queries, keys, k):
    q_n = jnp.linalg.norm(queries, axis=1, keepdims=True)
    k_n = jnp.linalg.norm(keys, axis=1, keepdims=True)
    qn = queries / jnp.maximum(q_n, 1e-8)
    kn = keys / jnp.maximum(k_n, 1e-8)
    kn_p = jnp.pad(kn, ((0, KP - N_KEYS), (0, 0)))
    sim = _sim_matrix(qn, kn_p)

    cand_vals, cand_idx = _sc_select(sim.reshape(-1))
    cand_vals = cand_vals.reshape(N_QUERIES, OUT_C)
    cand_idx = cand_idx.reshape(N_QUERIES, OUT_C)

    vals, pos = jax.lax.top_k(cand_vals, TOPK)
    idx = jnp.take_along_axis(cand_idx, pos, axis=1)
    return vals, idx


# speedup vs baseline: 6.9596x; 1.4835x over previous
import functools

import jax
import jax.numpy as jnp
from jax import lax
from jax.experimental import pallas as pl
from jax.experimental.pallas import tpu as pltpu
from jax.experimental.pallas import tpu_sc as plsc

N_QUERIES = 4096
N_KEYS = 100000
DIM = 128
TOPK = 100

BM = 1024
BN = 1024
KP = 100352

NW = 32
ROWS_PER_W = N_QUERIES // NW
NVEC = N_KEYS // 16
WVEC = 256
WARM = WVEC * 16
CAP = 6144
OUT_C = 512
PAD_VAL = -3.0
BS_T = 13
BS_T2 = 11


def _matmul_body(q_ref, k_ref, o_ref):
    o_ref[...] = jax.lax.dot_general(
        q_ref[...], k_ref[...],
        dimension_numbers=(((1,), (1,)), ((), ())),
        preferred_element_type=jnp.float32,
    )


def _sim_matrix(qn, kn):
    return pl.pallas_call(
        _matmul_body,
        grid=(N_QUERIES // BM, KP // BN),
        in_specs=[
            pl.BlockSpec((BM, DIM), lambda i, j: (i, 0)),
            pl.BlockSpec((BN, DIM), lambda i, j: (j, 0)),
        ],
        out_specs=pl.BlockSpec((BM, BN), lambda i, j: (i, j)),
        out_shape=jax.ShapeDtypeStruct((N_QUERIES, KP), jnp.float32),
    )(qn, kn)


def _splat_f32(x):
    return jnp.full((16,), x, jnp.float32)


def _splat_total(x):
    s = plsc.cumsum(x)
    return plsc.cummax(lax.rev(s, (0,)))


def _thresh_search(readers, lo0, hi0, iters):
    def bs_body(_, lohi):
        lo, hi = lohi
        mid = 0.5 * (lo + hi)
        acc = jnp.zeros((16,), jnp.int32)
        for read_vec, nvec in readers:
            def cnt_body(j, a, read_vec=read_vec):
                m = read_vec(j) >= mid
                return a + m.astype(jnp.int32)
            acc = lax.fori_loop(0, nvec, cnt_body, acc)
        ok = _splat_total(acc) >= TOPK
        return jnp.where(ok, mid, lo), jnp.where(ok, hi, mid)
    lo, hi = lax.fori_loop(0, iters, bs_body, (lo0, hi0))
    return lo


def _sc_select_body(sim_ref, vals_ref, idx_ref, row_v, cv, ci, ov, oi):
    cid = lax.axis_index("c")
    sid = lax.axis_index("s")
    wid = sid * 2 + cid
    iota = lax.iota(jnp.int32, 16)

    def do_row(r, _):
        row = wid * ROWS_PER_W + r
        pltpu.sync_copy(sim_ref.at[pl.ds(row * KP, N_KEYS)], row_v)

        t = _thresh_search([(lambda j: row_v[pl.ds(j * 16, 16)], WVEC)],
                           _splat_f32(-1.5), _splat_f32(1.5), BS_T)

        def filt_body(i, curv16):
            v = row_v[pl.ds(i * 16, 16)]
            m = v >= t
            pos = jnp.minimum(curv16, CAP - 16) + iota
            plsc.store_scatter(ci, [pos], iota + i * 16, mask=m)
            return curv16 + (m.astype(jnp.int32) << 4)
        curv16 = lax.fori_loop(WVEC, NVEC, filt_body,
                               jnp.zeros((16,), jnp.int32))

        def gath_body(j, _):
            ix = ci[pl.ds(j * 16, 16)]
            ixc = jnp.clip(ix, 0, N_KEYS - 1)
            v = plsc.load_gather(row_v, [ixc])
            valid = (j * 16) < curv16
            cv[pl.ds(j * 16, 16)] = jnp.where(valid, v, _splat_f32(PAD_VAL))
            return 0
        lax.fori_loop(0, CAP // 16, gath_body, 0)

        t2 = _thresh_search([(lambda j: row_v[pl.ds(j * 16, 16)], WVEC),
                             (lambda j: cv[pl.ds(j * 16, 16)], CAP // 16)],
                            t, _splat_f32(1.5), BS_T2)

        def oclr_body(j, _):
            ov[pl.ds(j * 16, 16)] = _splat_f32(PAD_VAL)
            return 0
        lax.fori_loop(0, OUT_C // 16, oclr_body, 0)

        def comp_warm(j, ocur16):
            v = row_v[pl.ds(j * 16, 16)]
            m = v >= t2
            pos = jnp.minimum(ocur16, OUT_C - 16) + iota
            plsc.store_scatter(ov, [pos], v, mask=m)
            plsc.store_scatter(oi, [pos], iota + j * 16, mask=m)
            return ocur16 + (m.astype(jnp.int32) << 4)
        ocur16 = lax.fori_loop(0, WVEC, comp_warm, jnp.zeros((16,), jnp.int32))

        def comp_app(j, ocur16):
            v = cv[pl.ds(j * 16, 16)]
            ix = ci[pl.ds(j * 16, 16)]
            m = v >= t2
            pos = jnp.minimum(ocur16, OUT_C - 16) + iota
            plsc.store_scatter(ov, [pos], v, mask=m)
            plsc.store_scatter(oi, [pos], ix, mask=m)
            return ocur16 + (m.astype(jnp.int32) << 4)
        lax.fori_loop(0, CAP // 16, comp_app, ocur16)

        pltpu.sync_copy(ov.at[pl.ds(0, OUT_C)],
                        vals_ref.at[pl.ds(row * OUT_C, OUT_C)])
        pltpu.sync_copy(oi.at[pl.ds(0, OUT_C)],
                        idx_ref.at[pl.ds(row * OUT_C, OUT_C)])
        return 0

    lax.fori_loop(0, ROWS_PER_W, do_row, 0)


@functools.partial(
    pl.kernel,
    out_type=[
        jax.ShapeDtypeStruct((N_QUERIES * OUT_C,), jnp.float32),
        jax.ShapeDtypeStruct((N_QUERIES * OUT_C,), jnp.int32),
    ],
    mesh=plsc.VectorSubcoreMesh(
        core_axis_name="c", subcore_axis_name="s",
        num_cores=2, num_subcores=16),
    compiler_params=pltpu.CompilerParams(needs_layout_passes=False),
    scratch_types=[
        pltpu.VMEM((N_KEYS,), jnp.float32),
        pltpu.VMEM((CAP,), jnp.float32),
        pltpu.VMEM((CAP,), jnp.int32),
        pltpu.VMEM((OUT_C,), jnp.float32),
        pltpu.VMEM((OUT_C,), jnp.int32),
    ],
)
def _sc_select(sim_ref, vals_ref, idx_ref, row_v, cv, ci, ov, oi):
    _sc_select_body(sim_ref, vals_ref, idx_ref, row_v, cv, ci, ov, oi)


def kernel(queries, keys, k):
    q_n = jnp.linalg.norm(queries, axis=1, keepdims=True)
    k_n = jnp.linalg.norm(keys, axis=1, keepdims=True)
    qn = queries / jnp.maximum(q_n, 1e-8)
    kn = keys / jnp.maximum(k_n, 1e-8)
    kn_p = jnp.pad(kn, ((0, KP - N_KEYS), (0, 0)))
    sim = _sim_matrix(qn, kn_p)

    cand_vals, cand_idx = _sc_select(sim.reshape(-1))
    cand_vals = cand_vals.reshape(N_QUERIES, OUT_C)
    cand_idx = cand_idx.reshape(N_QUERIES, OUT_C)

    neg_sorted, idx_sorted = jax.lax.sort(
        (-cand_vals, cand_idx), dimension=1, num_keys=2)
    return -neg_sorted[:, :TOPK], idx_sorted[:, :TOPK]


# speedup vs baseline: 8.7589x; 1.2585x over previous
import functools

import jax
import jax.numpy as jnp
from jax import lax
from jax.experimental import pallas as pl
from jax.experimental.pallas import tpu as pltpu
from jax.experimental.pallas import tpu_sc as plsc

N_QUERIES = 4096
N_KEYS = 100000
DIM = 128
TOPK = 100

BM = 1024
BN = 1024
KP = 100352

NW = 32
ROWS_PER_W = N_QUERIES // NW
NVEC = N_KEYS // 16
WVEC = 256
WARM = WVEC * 16
CAP = 6144
OUT_C = 512
PAD_VAL = -3.0
BS_T = 13
BS_T2 = 11


def _matmul_body(q_ref, k_ref, o_ref):
    o_ref[...] = jax.lax.dot_general(
        q_ref[...], k_ref[...],
        dimension_numbers=(((1,), (1,)), ((), ())),
        preferred_element_type=jnp.float32,
    )


def _sim_matrix(qn, kn):
    return pl.pallas_call(
        _matmul_body,
        grid=(N_QUERIES // BM, KP // BN),
        in_specs=[
            pl.BlockSpec((BM, DIM), lambda i, j: (i, 0)),
            pl.BlockSpec((BN, DIM), lambda i, j: (j, 0)),
        ],
        out_specs=pl.BlockSpec((BM, BN), lambda i, j: (i, j)),
        out_shape=jax.ShapeDtypeStruct((N_QUERIES, KP), jnp.float32),
    )(qn, kn)


def _splat_f32(x):
    return jnp.full((16,), x, jnp.float32)


def _splat_total(x):
    s = plsc.cumsum(x)
    return plsc.cummax(lax.rev(s, (0,)))


def _thresh_search(readers, lo0, hi0, iters):
    def bs_body(_, lohi):
        lo, hi = lohi
        mid = 0.5 * (lo + hi)
        acc = jnp.zeros((16,), jnp.int32)
        for read_vec, nvec in readers:
            assert nvec % 4 == 0
            def cnt_body(j, a, read_vec=read_vec):
                for u in range(4):
                    a = a + (read_vec(j * 4 + u) >= mid).astype(jnp.int32)
                return a
            acc = lax.fori_loop(0, nvec // 4, cnt_body, acc)
        ok = _splat_total(acc) >= TOPK
        return jnp.where(ok, mid, lo), jnp.where(ok, hi, mid)
    lo, hi = lax.fori_loop(0, iters, bs_body, (lo0, hi0))
    return lo


def _sc_select_body(sim_ref, vals_ref, idx_ref, row_v, cv, ci, ov, oi):
    cid = lax.axis_index("c")
    sid = lax.axis_index("s")
    wid = sid * 2 + cid
    iota = lax.iota(jnp.int32, 16)

    def do_row(r, _):
        row = wid * ROWS_PER_W + r
        pltpu.sync_copy(sim_ref.at[pl.ds(row * KP, N_KEYS)], row_v)

        t = _thresh_search([(lambda j: row_v[pl.ds(j * 16, 16)], WVEC)],
                           _splat_f32(-1.5), _splat_f32(1.5), BS_T)

        def filt_body(i2, curv16):
            for u in range(2):
                i = WVEC + i2 * 2 + u
                v = row_v[pl.ds(i * 16, 16)]
                m = v >= t
                pos = jnp.minimum(curv16, CAP - 16) + iota
                plsc.store_scatter(ci, [pos], iota + i * 16, mask=m)
                curv16 = curv16 + (m.astype(jnp.int32) << 4)
            return curv16
        curv16 = lax.fori_loop(0, (NVEC - WVEC) // 2, filt_body,
                               jnp.zeros((16,), jnp.int32))

        def gath_body(j4, _):
            for u in range(4):
                j = j4 * 4 + u
                ix = ci[pl.ds(j * 16, 16)]
                ixc = jnp.clip(ix, 0, N_KEYS - 1)
                v = plsc.load_gather(row_v, [ixc])
                valid = (j * 16) < curv16
                cv[pl.ds(j * 16, 16)] = jnp.where(valid, v, _splat_f32(PAD_VAL))
            return 0
        lax.fori_loop(0, CAP // 64, gath_body, 0)

        t2 = _thresh_search([(lambda j: row_v[pl.ds(j * 16, 16)], WVEC),
                             (lambda j: cv[pl.ds(j * 16, 16)], CAP // 16)],
                            t, _splat_f32(1.5), BS_T2)

        def oclr_body(j, _):
            ov[pl.ds(j * 16, 16)] = _splat_f32(PAD_VAL)
            return 0
        lax.fori_loop(0, OUT_C // 16, oclr_body, 0)

        def comp_warm(j4, ocur16):
            for u in range(4):
                j = j4 * 4 + u
                v = row_v[pl.ds(j * 16, 16)]
                m = v >= t2
                pos = jnp.minimum(ocur16, OUT_C - 16) + iota
                plsc.store_scatter(ov, [pos], v, mask=m)
                plsc.store_scatter(oi, [pos], iota + j * 16, mask=m)
                ocur16 = ocur16 + (m.astype(jnp.int32) << 4)
            return ocur16
        ocur16 = lax.fori_loop(0, WVEC // 4, comp_warm,
                               jnp.zeros((16,), jnp.int32))

        def comp_app(j4, ocur16):
            for u in range(4):
                j = j4 * 4 + u
                v = cv[pl.ds(j * 16, 16)]
                ix = ci[pl.ds(j * 16, 16)]
                m = v >= t2
                pos = jnp.minimum(ocur16, OUT_C - 16) + iota
                plsc.store_scatter(ov, [pos], v, mask=m)
                plsc.store_scatter(oi, [pos], ix, mask=m)
                ocur16 = ocur16 + (m.astype(jnp.int32) << 4)
            return ocur16
        lax.fori_loop(0, CAP // 64, comp_app, ocur16)

        pltpu.sync_copy(ov.at[pl.ds(0, OUT_C)],
                        vals_ref.at[pl.ds(row * OUT_C, OUT_C)])
        pltpu.sync_copy(oi.at[pl.ds(0, OUT_C)],
                        idx_ref.at[pl.ds(row * OUT_C, OUT_C)])
        return 0

    lax.fori_loop(0, ROWS_PER_W, do_row, 0)


@functools.partial(
    pl.kernel,
    out_type=[
        jax.ShapeDtypeStruct((N_QUERIES * OUT_C,), jnp.float32),
        jax.ShapeDtypeStruct((N_QUERIES * OUT_C,), jnp.int32),
    ],
    mesh=plsc.VectorSubcoreMesh(
        core_axis_name="c", subcore_axis_name="s",
        num_cores=2, num_subcores=16),
    compiler_params=pltpu.CompilerParams(needs_layout_passes=False),
    scratch_types=[
        pltpu.VMEM((N_KEYS,), jnp.float32),
        pltpu.VMEM((CAP,), jnp.float32),
        pltpu.VMEM((CAP,), jnp.int32),
        pltpu.VMEM((OUT_C,), jnp.float32),
        pltpu.VMEM((OUT_C,), jnp.int32),
    ],
)
def _sc_select(sim_ref, vals_ref, idx_ref, row_v, cv, ci, ov, oi):
    _sc_select_body(sim_ref, vals_ref, idx_ref, row_v, cv, ci, ov, oi)


def kernel(queries, keys, k):
    q_n = jnp.linalg.norm(queries, axis=1, keepdims=True)
    k_n = jnp.linalg.norm(keys, axis=1, keepdims=True)
    qn = queries / jnp.maximum(q_n, 1e-8)
    kn = keys / jnp.maximum(k_n, 1e-8)
    kn_p = jnp.pad(kn, ((0, KP - N_KEYS), (0, 0)))
    sim = _sim_matrix(qn, kn_p)

    cand_vals, cand_idx = _sc_select(sim.reshape(-1))
    cand_vals = cand_vals.reshape(N_QUERIES, OUT_C)
    cand_idx = cand_idx.reshape(N_QUERIES, OUT_C)

    neg_sorted, idx_sorted = jax.lax.sort(
        (-cand_vals, cand_idx), dimension=1, num_keys=2)
    return -neg_sorted[:, :TOPK], idx_sorted[:, :TOPK]


# speedup vs baseline: 9.6727x; 1.1043x over previous
import functools

import jax
import jax.numpy as jnp
from jax import lax
from jax.experimental import pallas as pl
from jax.experimental.pallas import tpu as pltpu
from jax.experimental.pallas import tpu_sc as plsc

N_QUERIES = 4096
N_KEYS = 100000
DIM = 128
TOPK = 100

BM = 1024
BN = 1024
KP = 100352

NW = 32
ROWS_PER_W = N_QUERIES // NW
NVEC = N_KEYS // 16
WVEC = 256
WARM = WVEC * 16
CAP = 6144
OUT_C = 512
PAD_VAL = -3.0
BS_T = 11
BS_T2 = 9


def _matmul_body(q_ref, k_ref, o_ref):
    o_ref[...] = jax.lax.dot_general(
        q_ref[...], k_ref[...],
        dimension_numbers=(((1,), (1,)), ((), ())),
        preferred_element_type=jnp.float32,
    )


def _sim_matrix(qn, kn):
    return pl.pallas_call(
        _matmul_body,
        grid=(N_QUERIES // BM, KP // BN),
        in_specs=[
            pl.BlockSpec((BM, DIM), lambda i, j: (i, 0)),
            pl.BlockSpec((BN, DIM), lambda i, j: (j, 0)),
        ],
        out_specs=pl.BlockSpec((BM, BN), lambda i, j: (i, j)),
        out_shape=jax.ShapeDtypeStruct((N_QUERIES, KP), jnp.float32),
    )(qn, kn)


def _splat_f32(x):
    return jnp.full((16,), x, jnp.float32)


def _splat_total(x):
    s = plsc.cumsum(x)
    return plsc.cummax(lax.rev(s, (0,)))


def _thresh_search(readers, lo0, hi0, iters):
    def bs_body(_, lohi):
        lo, hi = lohi
        mid = 0.5 * (lo + hi)
        acc = jnp.zeros((16,), jnp.int32)
        for read_vec, nvec in readers:
            assert nvec % 8 == 0
            def cnt_body(j, a, read_vec=read_vec):
                for u in range(8):
                    a = a + (read_vec(j * 8 + u) >= mid).astype(jnp.int32)
                return a
            acc = lax.fori_loop(0, nvec // 8, cnt_body, acc)
        ok = _splat_total(acc) >= TOPK
        return jnp.where(ok, mid, lo), jnp.where(ok, hi, mid)
    lo, hi = lax.fori_loop(0, iters, bs_body, (lo0, hi0))
    return lo


def _sc_select_body(sim_ref, vals_ref, idx_ref, row_v, cv, ci, ov, oi):
    cid = lax.axis_index("c")
    sid = lax.axis_index("s")
    wid = sid * 2 + cid
    iota = lax.iota(jnp.int32, 16)

    def do_row(r, _):
        row = wid * ROWS_PER_W + r
        pltpu.sync_copy(sim_ref.at[pl.ds(row * KP, N_KEYS)], row_v)

        t = _thresh_search([(lambda j: row_v[pl.ds(j * 16, 16)], WVEC)],
                           _splat_f32(-1.5), _splat_f32(1.5), BS_T)

        assert (NVEC - WVEC) % 6 == 0
        def filt_body(i6, curv16):
            for u in range(6):
                i = WVEC + i6 * 6 + u
                v = row_v[pl.ds(i * 16, 16)]
                m = v >= t
                pos = jnp.minimum(curv16, CAP - 16) + iota
                plsc.store_scatter(ci, [pos], iota + i * 16, mask=m)
                curv16 = curv16 + (m.astype(jnp.int32) << 4)
            return curv16
        curv16 = lax.fori_loop(0, (NVEC - WVEC) // 6, filt_body,
                               jnp.zeros((16,), jnp.int32))

        def gath_body(j4, _):
            for u in range(4):
                j = j4 * 4 + u
                ix = ci[pl.ds(j * 16, 16)]
                ixc = jnp.clip(ix, 0, N_KEYS - 1)
                v = plsc.load_gather(row_v, [ixc])
                valid = (j * 16) < curv16
                cv[pl.ds(j * 16, 16)] = jnp.where(valid, v, _splat_f32(PAD_VAL))
            return 0
        lax.fori_loop(0, CAP // 64, gath_body, 0)

        t2 = _thresh_search([(lambda j: row_v[pl.ds(j * 16, 16)], WVEC),
                             (lambda j: cv[pl.ds(j * 16, 16)], CAP // 16)],
                            t, _splat_f32(1.5), BS_T2)

        def oclr_body(j, _):
            ov[pl.ds(j * 16, 16)] = _splat_f32(PAD_VAL)
            return 0
        lax.fori_loop(0, OUT_C // 16, oclr_body, 0)

        def comp_warm(j4, ocur16):
            for u in range(4):
                j = j4 * 4 + u
                v = row_v[pl.ds(j * 16, 16)]
                m = v >= t2
                pos = jnp.minimum(ocur16, OUT_C - 16) + iota
                plsc.store_scatter(ov, [pos], v, mask=m)
                plsc.store_scatter(oi, [pos], iota + j * 16, mask=m)
                ocur16 = ocur16 + (m.astype(jnp.int32) << 4)
            return ocur16
        ocur16 = lax.fori_loop(0, WVEC // 4, comp_warm,
                               jnp.zeros((16,), jnp.int32))

        def comp_app(j4, ocur16):
            for u in range(4):
                j = j4 * 4 + u
                v = cv[pl.ds(j * 16, 16)]
                ix = ci[pl.ds(j * 16, 16)]
                m = v >= t2
                pos = jnp.minimum(ocur16, OUT_C - 16) + iota
                plsc.store_scatter(ov, [pos], v, mask=m)
                plsc.store_scatter(oi, [pos], ix, mask=m)
                ocur16 = ocur16 + (m.astype(jnp.int32) << 4)
            return ocur16
        lax.fori_loop(0, CAP // 64, comp_app, ocur16)

        pltpu.sync_copy(ov.at[pl.ds(0, OUT_C)],
                        vals_ref.at[pl.ds(row * OUT_C, OUT_C)])
        pltpu.sync_copy(oi.at[pl.ds(0, OUT_C)],
                        idx_ref.at[pl.ds(row * OUT_C, OUT_C)])
        return 0

    lax.fori_loop(0, ROWS_PER_W, do_row, 0)


@functools.partial(
    pl.kernel,
    out_type=[
        jax.ShapeDtypeStruct((N_QUERIES * OUT_C,), jnp.float32),
        jax.ShapeDtypeStruct((N_QUERIES * OUT_C,), jnp.int32),
    ],
    mesh=plsc.VectorSubcoreMesh(
        core_axis_name="c", subcore_axis_name="s",
        num_cores=2, num_subcores=16),
    compiler_params=pltpu.CompilerParams(needs_layout_passes=False),
    scratch_types=[
        pltpu.VMEM((N_KEYS,), jnp.float32),
        pltpu.VMEM((CAP,), jnp.float32),
        pltpu.VMEM((CAP,), jnp.int32),
        pltpu.VMEM((OUT_C,), jnp.float32),
        pltpu.VMEM((OUT_C,), jnp.int32),
    ],
)
def _sc_select(sim_ref, vals_ref, idx_ref, row_v, cv, ci, ov, oi):
    _sc_select_body(sim_ref, vals_ref, idx_ref, row_v, cv, ci, ov, oi)


def kernel(queries, keys, k):
    q_n = jnp.linalg.norm(queries, axis=1, keepdims=True)
    k_n = jnp.linalg.norm(keys, axis=1, keepdims=True)
    qn = queries / jnp.maximum(q_n, 1e-8)
    kn = keys / jnp.maximum(k_n, 1e-8)
    kn_p = jnp.pad(kn, ((0, KP - N_KEYS), (0, 0)))
    sim = _sim_matrix(qn, kn_p)

    cand_vals, cand_idx = _sc_select(sim.reshape(-1))
    cand_vals = cand_vals.reshape(N_QUERIES, OUT_C)
    cand_idx = cand_idx.reshape(N_QUERIES, OUT_C)

    neg_sorted, idx_sorted = jax.lax.sort(
        (-cand_vals, cand_idx), dimension=1, num_keys=2)
    return -neg_sorted[:, :TOPK], idx_sorted[:, :TOPK]


# speedup vs baseline: 10.3372x; 1.0687x over previous
import functools

import jax
import jax.numpy as jnp
from jax import lax
from jax.experimental import pallas as pl
from jax.experimental.pallas import tpu as pltpu
from jax.experimental.pallas import tpu_sc as plsc

N_QUERIES = 4096
N_KEYS = 100000
DIM = 128
TOPK = 100

BM = 1024
BN = 1024
KP = 100352

NW = 32
ROWS_PER_W = N_QUERIES // NW
NVEC = N_KEYS // 16
WVEC = 256
WARM = WVEC * 16
CAP = 6144
OUT_C = 512
PAD_VAL = -3.0
BS_T = 11
BS_T2 = 9


def _matmul_body(q_ref, k_ref, o_ref):
    o_ref[...] = jax.lax.dot_general(
        q_ref[...], k_ref[...],
        dimension_numbers=(((1,), (1,)), ((), ())),
        preferred_element_type=jnp.float32,
    )


def _sim_matrix(qn, kn):
    return pl.pallas_call(
        _matmul_body,
        grid=(N_QUERIES // BM, KP // BN),
        in_specs=[
            pl.BlockSpec((BM, DIM), lambda i, j: (i, 0)),
            pl.BlockSpec((BN, DIM), lambda i, j: (j, 0)),
        ],
        out_specs=pl.BlockSpec((BM, BN), lambda i, j: (i, j)),
        out_shape=jax.ShapeDtypeStruct((N_QUERIES, KP), jnp.float32),
    )(qn, kn)


def _splat_f32(x):
    return jnp.full((16,), x, jnp.float32)


def _splat_total(x):
    s = plsc.cumsum(x)
    return plsc.cummax(lax.rev(s, (0,)))


def _thresh_search(readers, lo0, hi0, iters):
    def bs_body(_, lohi):
        lo, hi = lohi
        mid = 0.5 * (lo + hi)
        acc = jnp.zeros((16,), jnp.int32)
        for read_vec, ngroups in readers:
            def cnt_body(j, a, read_vec=read_vec):
                for u in range(8):
                    a = a + (read_vec(j * 8 + u) >= mid).astype(jnp.int32)
                return a
            acc = lax.fori_loop(0, ngroups, cnt_body, acc)
        ok = _splat_total(acc) >= TOPK
        return jnp.where(ok, mid, lo), jnp.where(ok, hi, mid)
    lo, hi = lax.fori_loop(0, iters, bs_body, (lo0, hi0))
    return lo


def _sc_select_body(sim_ref, vals_ref, idx_ref, row_v, cv, ci, ov, oi):
    cid = lax.axis_index("c")
    sid = lax.axis_index("s")
    wid = sid * 2 + cid
    iota = lax.iota(jnp.int32, 16)

    def do_row(r, _):
        row = wid * ROWS_PER_W + r
        pltpu.sync_copy(sim_ref.at[pl.ds(row * KP, N_KEYS)], row_v)

        t = _thresh_search([(lambda j: row_v[pl.ds(j * 16, 16)], WVEC // 8)],
                           _splat_f32(-1.5), _splat_f32(1.5), BS_T)

        assert (NVEC - WVEC) % 6 == 0
        def filt_body(i6, curv16):
            for u in range(6):
                i = WVEC + i6 * 6 + u
                v = row_v[pl.ds(i * 16, 16)]
                m = v >= t
                pos = jnp.minimum(curv16, CAP - 16) + iota
                plsc.store_scatter(ci, [pos], iota + i * 16, mask=m)
                curv16 = curv16 + (m.astype(jnp.int32) << 4)
            return curv16
        curv16 = lax.fori_loop(0, (NVEC - WVEC) // 6, filt_body,
                               jnp.zeros((16,), jnp.int32))

        mlc = jnp.minimum(jnp.max(curv16) >> 4, CAP // 16)
        nb8 = (mlc + 7) >> 3

        def gath_body(j4, _):
            for u in range(4):
                j = j4 * 4 + u
                ix = ci[pl.ds(j * 16, 16)]
                ixc = jnp.clip(ix, 0, N_KEYS - 1)
                v = plsc.load_gather(row_v, [ixc])
                valid = (j * 16) < curv16
                cv[pl.ds(j * 16, 16)] = jnp.where(valid, v, _splat_f32(PAD_VAL))
            return 0
        lax.fori_loop(0, nb8 * 2, gath_body, 0)

        t2 = _thresh_search([(lambda j: row_v[pl.ds(j * 16, 16)], WVEC // 8),
                             (lambda j: cv[pl.ds(j * 16, 16)], nb8)],
                            t, _splat_f32(1.5), BS_T2)

        def oclr_body(j, _):
            ov[pl.ds(j * 16, 16)] = _splat_f32(PAD_VAL)
            return 0
        lax.fori_loop(0, OUT_C // 16, oclr_body, 0)

        def comp_warm(j4, ocur16):
            for u in range(4):
                j = j4 * 4 + u
                v = row_v[pl.ds(j * 16, 16)]
                m = v >= t2
                pos = jnp.minimum(ocur16, OUT_C - 16) + iota
                plsc.store_scatter(ov, [pos], v, mask=m)
                plsc.store_scatter(oi, [pos], iota + j * 16, mask=m)
                ocur16 = ocur16 + (m.astype(jnp.int32) << 4)
            return ocur16
        ocur16 = lax.fori_loop(0, WVEC // 4, comp_warm,
                               jnp.zeros((16,), jnp.int32))

        def comp_app(j4, ocur16):
            for u in range(4):
                j = j4 * 4 + u
                v = cv[pl.ds(j * 16, 16)]
                ix = ci[pl.ds(j * 16, 16)]
                m = v >= t2
                pos = jnp.minimum(ocur16, OUT_C - 16) + iota
                plsc.store_scatter(ov, [pos], v, mask=m)
                plsc.store_scatter(oi, [pos], ix, mask=m)
                ocur16 = ocur16 + (m.astype(jnp.int32) << 4)
            return ocur16
        lax.fori_loop(0, nb8 * 2, comp_app, ocur16)

        pltpu.sync_copy(ov.at[pl.ds(0, OUT_C)],
                        vals_ref.at[pl.ds(row * OUT_C, OUT_C)])
        pltpu.sync_copy(oi.at[pl.ds(0, OUT_C)],
                        idx_ref.at[pl.ds(row * OUT_C, OUT_C)])
        return 0

    lax.fori_loop(0, ROWS_PER_W, do_row, 0)


@functools.partial(
    pl.kernel,
    out_type=[
        jax.ShapeDtypeStruct((N_QUERIES * OUT_C,), jnp.float32),
        jax.ShapeDtypeStruct((N_QUERIES * OUT_C,), jnp.int32),
    ],
    mesh=plsc.VectorSubcoreMesh(
        core_axis_name="c", subcore_axis_name="s",
        num_cores=2, num_subcores=16),
    compiler_params=pltpu.CompilerParams(needs_layout_passes=False),
    scratch_types=[
        pltpu.VMEM((N_KEYS,), jnp.float32),
        pltpu.VMEM((CAP,), jnp.float32),
        pltpu.VMEM((CAP,), jnp.int32),
        pltpu.VMEM((OUT_C,), jnp.float32),
        pltpu.VMEM((OUT_C,), jnp.int32),
    ],
)
def _sc_select(sim_ref, vals_ref, idx_ref, row_v, cv, ci, ov, oi):
    _sc_select_body(sim_ref, vals_ref, idx_ref, row_v, cv, ci, ov, oi)


def kernel(queries, keys, k):
    q_n = jnp.linalg.norm(queries, axis=1, keepdims=True)
    k_n = jnp.linalg.norm(keys, axis=1, keepdims=True)
    qn = queries / jnp.maximum(q_n, 1e-8)
    kn = keys / jnp.maximum(k_n, 1e-8)
    kn_p = jnp.pad(kn, ((0, KP - N_KEYS), (0, 0)))
    sim = _sim_matrix(qn, kn_p)

    cand_vals, cand_idx = _sc_select(sim.reshape(-1))
    cand_vals = cand_vals.reshape(N_QUERIES, OUT_C)
    cand_idx = cand_idx.reshape(N_QUERIES, OUT_C)

    neg_sorted, idx_sorted = jax.lax.sort(
        (-cand_vals, cand_idx), dimension=1, num_keys=2)
    return -neg_sorted[:, :TOPK], idx_sorted[:, :TOPK]


# speedup vs baseline: 10.4434x; 1.0103x over previous
import functools

import jax
import jax.numpy as jnp
from jax import lax
from jax.experimental import pallas as pl
from jax.experimental.pallas import tpu as pltpu
from jax.experimental.pallas import tpu_sc as plsc

N_QUERIES = 4096
N_KEYS = 100000
DIM = 128
TOPK = 100

BM = 1024
BN = 1024
KP = 100352

NW = 32
ROWS_PER_W = N_QUERIES // NW
NVEC = N_KEYS // 16
WVEC = 256
WARM = WVEC * 16
CAP = 6144
OUT_C = 512
PAD_VAL = -3.0
BS_T = 11
BS_T2 = 9


def _matmul_body(q_ref, k_ref, o_ref):
    o_ref[...] = jax.lax.dot_general(
        q_ref[...], k_ref[...],
        dimension_numbers=(((1,), (1,)), ((), ())),
        preferred_element_type=jnp.float32,
    )


def _sim_matrix(qn, kn):
    return pl.pallas_call(
        _matmul_body,
        grid=(N_QUERIES // BM, KP // BN),
        in_specs=[
            pl.BlockSpec((BM, DIM), lambda i, j: (i, 0)),
            pl.BlockSpec((BN, DIM), lambda i, j: (j, 0)),
        ],
        out_specs=pl.BlockSpec((BM, BN), lambda i, j: (i, j)),
        out_shape=jax.ShapeDtypeStruct((N_QUERIES, KP), jnp.float32),
    )(qn, kn)


def _splat_f32(x):
    return jnp.full((16,), x, jnp.float32)


def _splat_total(x):
    s = plsc.cumsum(x)
    return plsc.cummax(lax.rev(s, (0,)))


def _thresh_search(readers, lo0, hi0, iters):
    def bs_body(_, lohi):
        lo, hi = lohi
        mid = 0.5 * (lo + hi)
        acc = jnp.zeros((16,), jnp.int32)
        for read_vec, ngroups in readers:
            def cnt_body(j, a, read_vec=read_vec):
                for u in range(8):
                    a = a + (read_vec(j * 8 + u) >= mid).astype(jnp.int32)
                return a
            acc = lax.fori_loop(0, ngroups, cnt_body, acc)
        ok = _splat_total(acc) >= TOPK
        return jnp.where(ok, mid, lo), jnp.where(ok, hi, mid)
    lo, hi = lax.fori_loop(0, iters, bs_body, (lo0, hi0))
    return lo


def _sc_select_body(sim_ref, vals_ref, idx_ref, row_v, cv, ci, ov, oi):
    cid = lax.axis_index("c")
    sid = lax.axis_index("s")
    wid = sid * 2 + cid
    iota = lax.iota(jnp.int32, 16)

    def do_row(r, _):
        row = wid * ROWS_PER_W + r
        pltpu.sync_copy(sim_ref.at[pl.ds(row * KP, N_KEYS)], row_v)

        t = _thresh_search([(lambda j: row_v[pl.ds(j * 16, 16)], WVEC // 8)],
                           _splat_f32(-1.5), _splat_f32(1.5), BS_T)

        assert (NVEC - WVEC) % 9 == 0
        capv = jnp.full((16,), CAP - 16, jnp.int32) + iota
        def filt_body(i9, curvi):
            for u in range(9):
                i = WVEC + i9 * 9 + u
                v = row_v[pl.ds(i * 16, 16)]
                m = v >= t
                pos = jnp.minimum(curvi, capv)
                plsc.store_scatter(ci, [pos], iota + i * 16, mask=m)
                curvi = curvi + (m.astype(jnp.int32) << 4)
            return curvi
        curvi = lax.fori_loop(0, (NVEC - WVEC) // 9, filt_body, iota)
        curv16 = curvi - iota

        mlc = jnp.minimum(jnp.max(curv16) >> 4, CAP // 16)
        nb8 = (mlc + 7) >> 3

        def gath_body(j4, _):
            for u in range(4):
                j = j4 * 4 + u
                ix = ci[pl.ds(j * 16, 16)]
                ixc = jnp.clip(ix, 0, N_KEYS - 1)
                v = plsc.load_gather(row_v, [ixc])
                valid = (j * 16) < curv16
                cv[pl.ds(j * 16, 16)] = jnp.where(valid, v, _splat_f32(PAD_VAL))
            return 0
        lax.fori_loop(0, nb8 * 2, gath_body, 0)

        t2 = _thresh_search([(lambda j: row_v[pl.ds(j * 16, 16)], WVEC // 8),
                             (lambda j: cv[pl.ds(j * 16, 16)], nb8)],
                            t, _splat_f32(1.5), BS_T2)

        def oclr_body(j, _):
            ov[pl.ds(j * 16, 16)] = _splat_f32(PAD_VAL)
            return 0
        lax.fori_loop(0, OUT_C // 16, oclr_body, 0)

        def comp_warm(j4, ocur16):
            for u in range(4):
                j = j4 * 4 + u
                v = row_v[pl.ds(j * 16, 16)]
                m = v >= t2
                pos = jnp.minimum(ocur16, OUT_C - 16) + iota
                plsc.store_scatter(ov, [pos], v, mask=m)
                plsc.store_scatter(oi, [pos], iota + j * 16, mask=m)
                ocur16 = ocur16 + (m.astype(jnp.int32) << 4)
            return ocur16
        ocur16 = lax.fori_loop(0, WVEC // 4, comp_warm,
                               jnp.zeros((16,), jnp.int32))

        def comp_app(j4, ocur16):
            for u in range(4):
                j = j4 * 4 + u
                v = cv[pl.ds(j * 16, 16)]
                ix = ci[pl.ds(j * 16, 16)]
                m = v >= t2
                pos = jnp.minimum(ocur16, OUT_C - 16) + iota
                plsc.store_scatter(ov, [pos], v, mask=m)
                plsc.store_scatter(oi, [pos], ix, mask=m)
                ocur16 = ocur16 + (m.astype(jnp.int32) << 4)
            return ocur16
        lax.fori_loop(0, nb8 * 2, comp_app, ocur16)

        pltpu.sync_copy(ov.at[pl.ds(0, OUT_C)],
                        vals_ref.at[pl.ds(row * OUT_C, OUT_C)])
        pltpu.sync_copy(oi.at[pl.ds(0, OUT_C)],
                        idx_ref.at[pl.ds(row * OUT_C, OUT_C)])
        return 0

    lax.fori_loop(0, ROWS_PER_W, do_row, 0)


@functools.partial(
    pl.kernel,
    out_type=[
        jax.ShapeDtypeStruct((N_QUERIES * OUT_C,), jnp.float32),
        jax.ShapeDtypeStruct((N_QUERIES * OUT_C,), jnp.int32),
    ],
    mesh=plsc.VectorSubcoreMesh(
        core_axis_name="c", subcore_axis_name="s",
        num_cores=2, num_subcores=16),
    compiler_params=pltpu.CompilerParams(needs_layout_passes=False),
    scratch_types=[
        pltpu.VMEM((N_KEYS,), jnp.float32),
        pltpu.VMEM((CAP,), jnp.float32),
        pltpu.VMEM((CAP,), jnp.int32),
        pltpu.VMEM((OUT_C,), jnp.float32),
        pltpu.VMEM((OUT_C,), jnp.int32),
    ],
)
def _sc_select(sim_ref, vals_ref, idx_ref, row_v, cv, ci, ov, oi):
    _sc_select_body(sim_ref, vals_ref, idx_ref, row_v, cv, ci, ov, oi)


def kernel(queries, keys, k):
    q_n = jnp.linalg.norm(queries, axis=1, keepdims=True)
    k_n = jnp.linalg.norm(keys, axis=1, keepdims=True)
    qn = queries / jnp.maximum(q_n, 1e-8)
    kn = keys / jnp.maximum(k_n, 1e-8)
    kn_p = jnp.pad(kn, ((0, KP - N_KEYS), (0, 0)))
    sim = _sim_matrix(qn, kn_p)

    cand_vals, cand_idx = _sc_select(sim.reshape(-1))
    cand_vals = cand_vals.reshape(N_QUERIES, OUT_C)
    cand_idx = cand_idx.reshape(N_QUERIES, OUT_C)

    neg_sorted, idx_sorted = jax.lax.sort(
        (-cand_vals, cand_idx), dimension=1, num_keys=2)
    return -neg_sorted[:, :TOPK], idx_sorted[:, :TOPK]
